# initial kernel scaffold (unmeasured)
import jax
import jax.numpy as jnp
from jax import lax
from jax.experimental import pallas as pl
from jax.experimental.pallas import tpu as pltpu

N_DEV = 4
N_EXP = 16
E_PER_DEV = 4
CAP = 102


def kernel(x, router_W, route_idx, expert_W):
    del router_W
    n_tok, d_model = x.shape
    _, _, d_ff = expert_W.shape

    def body(x_ref, idx_ref, w_ref, out_ref, comm_ref, send_sems, recv_sems):
        my = lax.axis_index("i")
        left = lax.rem(my + N_DEV - 1, N_DEV)
        right = lax.rem(my + 1, N_DEV)

        barrier_sem = pltpu.get_barrier_semaphore()
        for nbr in (left, right):
            pl.semaphore_signal(
                barrier_sem, inc=1,
                device_id=(nbr,), device_id_type=pl.DeviceIdType.MESH,
            )
        pl.semaphore_wait(barrier_sem, 2)

        idx = idx_ref[:, :]
        e_ids = lax.broadcasted_iota(jnp.int32, (n_tok, N_EXP), 1)
        onehot = (idx == e_ids).astype(jnp.float32)
        cum = onehot
        s = 1
        while s < n_tok:
            cum = cum + jnp.concatenate(
                [jnp.zeros((s, N_EXP), jnp.float32), cum[:-s, :]], axis=0
            )
            s *= 2
        kept = jnp.sum(onehot * (cum <= CAP), axis=1, keepdims=True)

        acc = jnp.zeros((n_tok, d_ff), jnp.float32)
        for j in range(E_PER_DEV):
            mask = jnp.where(
                (idx == my * E_PER_DEV + j) & (kept > 0.5), 1.0, 0.0
            )
            acc = acc + jnp.dot(
                x_ref[:, :] * mask, w_ref[j], preferred_element_type=jnp.float32
            )
        out_ref[:, :] = acc
        comm_ref[0, :, :] = acc

        for h in range(N_DEV - 1):
            rdma = pltpu.make_async_remote_copy(
                src_ref=comm_ref.at[h],
                dst_ref=comm_ref.at[h + 1],
                send_sem=send_sems.at[h],
                recv_sem=recv_sems.at[h],
                device_id=(right,),
                device_id_type=pl.DeviceIdType.MESH,
            )
            rdma.start()
            rdma.wait()
            out_ref[:, :] = out_ref[:, :] + comm_ref[h + 1, :, :]

    return pl.pallas_call(
        body,
        out_shape=jax.ShapeDtypeStruct((n_tok, d_ff), jnp.float32),
        in_specs=[
            pl.BlockSpec(memory_space=pltpu.VMEM),
            pl.BlockSpec(memory_space=pltpu.VMEM),
            pl.BlockSpec(memory_space=pltpu.VMEM),
        ],
        out_specs=pl.BlockSpec(memory_space=pltpu.VMEM),
        scratch_shapes=[
            pltpu.VMEM((N_DEV, n_tok, d_ff), jnp.float32),
            pltpu.SemaphoreType.DMA((N_DEV - 1,)),
            pltpu.SemaphoreType.DMA((N_DEV - 1,)),
        ],
        compiler_params=pltpu.CompilerParams(
            collective_id=0,
            vmem_limit_bytes=100 * 1024 * 1024,
        ),
    )(x, route_idx, expert_W)


# baseline (device time: 177450 ns/iter reference)
import jax
import jax.numpy as jnp
from jax import lax
from jax.experimental import pallas as pl
from jax.experimental.pallas import tpu as pltpu

N_DEV = 4
N_EXP = 16
E_PER_DEV = 4
CAP = 102
BLK = 512


def kernel(x, router_W, route_idx, expert_W):
    del router_W
    n_tok, d_model = x.shape
    _, _, d_ff = expert_W.shape

    def body(x_ref, idx_ref, w_ref, out_ref, comm_ref, send_sems, recv_sems):
        my = lax.axis_index("i")
        left = lax.rem(my + N_DEV - 1, N_DEV)
        right = lax.rem(my + 1, N_DEV)

        barrier_sem = pltpu.get_barrier_semaphore()
        for nbr in (left, right):
            pl.semaphore_signal(
                barrier_sem, inc=1,
                device_id=(nbr,), device_id_type=pl.DeviceIdType.MESH,
            )
        pl.semaphore_wait(barrier_sem, 2)

        idx = idx_ref[:, :]
        e_ids = lax.broadcasted_iota(jnp.int32, (n_tok, N_EXP), 1)
        onehot = (idx == e_ids).astype(jnp.float32)
        cum = onehot
        s = 1
        while s < n_tok:
            cum = cum + jnp.concatenate(
                [jnp.zeros((s, N_EXP), jnp.float32), cum[:-s, :]], axis=0
            )
            s *= 2
        kept = jnp.sum(onehot * (cum <= CAP), axis=1, keepdims=True)

        acc = jnp.zeros((n_tok, d_ff), jnp.float32)
        for j in range(E_PER_DEV):
            mask = jnp.where(
                (idx == my * E_PER_DEV + j) & (kept > 0.5), 1.0, 0.0
            )
            acc = acc + jnp.dot(
                x_ref[:, :] * mask, w_ref[j], preferred_element_type=jnp.float32
            )
        out_ref[:, :] = acc

        def blk(b):
            return pl.ds(b * BLK, BLK)

        for h in range(N_DEV - 1):
            sb = lax.rem(my - h + N_DEV, N_DEV)
            rb = lax.rem(my - 1 - h + 2 * N_DEV, N_DEV)
            rdma = pltpu.make_async_remote_copy(
                src_ref=out_ref.at[blk(sb), :],
                dst_ref=comm_ref.at[h],
                send_sem=send_sems.at[h],
                recv_sem=recv_sems.at[h],
                device_id=(right,),
                device_id_type=pl.DeviceIdType.MESH,
            )
            rdma.start()
            rdma.wait()
            out_ref[blk(rb), :] = out_ref[blk(rb), :] + comm_ref[h, :, :]

        for h in range(N_DEV - 1):
            gs = lax.rem(my + 1 - h + N_DEV, N_DEV)
            gr = lax.rem(my - h + N_DEV, N_DEV)
            k = N_DEV - 1 + h
            rdma = pltpu.make_async_remote_copy(
                src_ref=out_ref.at[blk(gs), :],
                dst_ref=comm_ref.at[k],
                send_sem=send_sems.at[k],
                recv_sem=recv_sems.at[k],
                device_id=(right,),
                device_id_type=pl.DeviceIdType.MESH,
            )
            rdma.start()
            rdma.wait()
            out_ref[blk(gr), :] = comm_ref[k, :, :]

    return pl.pallas_call(
        body,
        out_shape=jax.ShapeDtypeStruct((n_tok, d_ff), jnp.float32),
        in_specs=[
            pl.BlockSpec(memory_space=pltpu.VMEM),
            pl.BlockSpec(memory_space=pltpu.VMEM),
            pl.BlockSpec(memory_space=pltpu.VMEM),
        ],
        out_specs=pl.BlockSpec(memory_space=pltpu.VMEM),
        scratch_shapes=[
            pltpu.VMEM((2 * (N_DEV - 1), BLK, d_ff), jnp.float32),
            pltpu.SemaphoreType.DMA((2 * (N_DEV - 1),)),
            pltpu.SemaphoreType.DMA((2 * (N_DEV - 1),)),
        ],
        compiler_params=pltpu.CompilerParams(
            collective_id=0,
            vmem_limit_bytes=100 * 1024 * 1024,
        ),
    )(x, route_idx, expert_W)


# device time: 101790 ns/iter; 1.7433x vs baseline; 1.7433x over previous
import jax
import jax.numpy as jnp
from jax import lax
from jax.experimental import pallas as pl
from jax.experimental.pallas import tpu as pltpu

N_DEV = 4
N_EXP = 16
E_PER_DEV = 4
CAP = 102
BLK = 512


def kernel(x, router_W, route_idx, expert_W):
    del router_W
    n_tok, d_model = x.shape
    _, _, d_ff = expert_W.shape
    half = d_ff // 2
    CW, CCW = slice(0, half), slice(half, d_ff)

    def body(x_ref, idx_ref, w_ref, out_ref, kept_ref, cw_buf, ccw_buf,
             cw_ssem, cw_rsem, ccw_ssem, ccw_rsem):
        my = lax.axis_index("i")
        left = lax.rem(my + N_DEV - 1, N_DEV)
        right = lax.rem(my + 1, N_DEV)

        barrier_sem = pltpu.get_barrier_semaphore()
        for nbr in (left, right):
            pl.semaphore_signal(
                barrier_sem, inc=1,
                device_id=(nbr,), device_id_type=pl.DeviceIdType.MESH,
            )
        pl.semaphore_wait(barrier_sem, 2)

        idx = idx_ref[:, :]
        e_ids = lax.broadcasted_iota(jnp.int32, (n_tok, N_EXP), 1)
        onehot = (idx == e_ids).astype(jnp.float32)
        cum = onehot
        s = 1
        while s < n_tok:
            cum = cum + jnp.concatenate(
                [jnp.zeros((s, N_EXP), jnp.float32), cum[:-s, :]], axis=0
            )
            s *= 2
        kept_ref[:, :] = jnp.sum(
            onehot * (cum <= CAP), axis=1, keepdims=True
        )

        def rows(boff):
            b = lax.rem(my + boff + 2 * N_DEV, N_DEV)
            return pl.ds(b * BLK, BLK)

        def compute_block(boff):
            b = lax.rem(my + boff + 2 * N_DEV, N_DEV)
            r = pl.ds(b * BLK, BLK)
            xb = x_ref[r, :]
            idx_b = idx_ref[r, :]
            kept_b = kept_ref[r, :]
            acc = jnp.zeros((BLK, d_ff), jnp.float32)
            for j in range(E_PER_DEV):
                m = jnp.where(
                    (idx_b == my * E_PER_DEV + j) & (kept_b > 0.5), 1.0, 0.0
                )
                acc = acc + jnp.dot(
                    xb * m, w_ref[j], preferred_element_type=jnp.float32
                )
            out_ref[r, :] = acc

        def rs_desc(h, send_boff):
            cw = pltpu.make_async_remote_copy(
                src_ref=out_ref.at[rows(send_boff), CW],
                dst_ref=cw_buf.at[h],
                send_sem=cw_ssem.at[h], recv_sem=cw_rsem.at[h],
                device_id=(right,), device_id_type=pl.DeviceIdType.MESH,
            )
            ccw = pltpu.make_async_remote_copy(
                src_ref=out_ref.at[rows(-send_boff), CCW],
                dst_ref=ccw_buf.at[h],
                send_sem=ccw_ssem.at[h], recv_sem=ccw_rsem.at[h],
                device_id=(left,), device_id_type=pl.DeviceIdType.MESH,
            )
            return cw, ccw

        def rs_accum(h, recv_boff):
            r_cw, r_ccw = rows(recv_boff), rows(-recv_boff)
            out_ref[r_cw, CW] = out_ref[r_cw, CW] + cw_buf[h]
            out_ref[r_ccw, CCW] = out_ref[r_ccw, CCW] + ccw_buf[h]

        def ag_desc(h, send_boff):
            k = N_DEV - 1 + h
            cw = pltpu.make_async_remote_copy(
                src_ref=out_ref.at[rows(send_boff), CW],
                dst_ref=out_ref.at[rows(send_boff), CW],
                send_sem=cw_ssem.at[k], recv_sem=cw_rsem.at[k],
                device_id=(right,), device_id_type=pl.DeviceIdType.MESH,
            )
            ccw = pltpu.make_async_remote_copy(
                src_ref=out_ref.at[rows(-send_boff), CCW],
                dst_ref=out_ref.at[rows(-send_boff), CCW],
                send_sem=ccw_ssem.at[k], recv_sem=ccw_rsem.at[k],
                device_id=(left,), device_id_type=pl.DeviceIdType.MESH,
            )
            return cw, ccw

        compute_block(0)

        cw0, ccw0 = rs_desc(0, 0)
        cw0.start()
        ccw0.start()
        compute_block(1)
        compute_block(-1)
        cw0.wait()
        ccw0.wait()
        rs_accum(0, -1)

        cw1, ccw1 = rs_desc(1, -1)
        cw1.start()
        ccw1.start()
        compute_block(2)
        cw1.wait()
        ccw1.wait()
        rs_accum(1, -2)

        cw2, ccw2 = rs_desc(2, 2)
        cw2.start()
        ccw2.start()
        cw2.wait()
        ccw2.wait()
        rs_accum(2, -3)

        for h, send_boff in enumerate((1, 0, -1)):
            cw_h, ccw_h = ag_desc(h, send_boff)
            cw_h.start()
            ccw_h.start()
            cw_h.wait()
            ccw_h.wait()

    grid_spec = None
    return pl.pallas_call(
        body,
        out_shape=jax.ShapeDtypeStruct((n_tok, d_ff), jnp.float32),
        in_specs=[
            pl.BlockSpec(memory_space=pltpu.VMEM),
            pl.BlockSpec(memory_space=pltpu.VMEM),
            pl.BlockSpec(memory_space=pltpu.VMEM),
        ],
        out_specs=pl.BlockSpec(memory_space=pltpu.VMEM),
        scratch_shapes=[
            pltpu.VMEM((n_tok, 1), jnp.float32),
            pltpu.VMEM((N_DEV - 1, BLK, half), jnp.float32),
            pltpu.VMEM((N_DEV - 1, BLK, half), jnp.float32),
            pltpu.SemaphoreType.DMA((2 * (N_DEV - 1),)),
            pltpu.SemaphoreType.DMA((2 * (N_DEV - 1),)),
            pltpu.SemaphoreType.DMA((2 * (N_DEV - 1),)),
            pltpu.SemaphoreType.DMA((2 * (N_DEV - 1),)),
        ],
        compiler_params=pltpu.CompilerParams(
            collective_id=0,
            vmem_limit_bytes=100 * 1024 * 1024,
        ),
    )(x, route_idx, expert_W)
